# Initial kernel scaffold; baseline (speedup 1.0000x reference)
import jax
import jax.numpy as jnp
from jax.experimental import pallas as pl


def kernel(x, edge_index, W1, b1, W2, b2):
    raise NotImplementedError("write your pallas kernel here")


# speedup vs baseline: 22.8725x; 22.8725x over previous
import functools

import jax
import jax.numpy as jnp
from jax import lax
from jax.experimental import pallas as pl
from jax.experimental.pallas import tpu as pltpu
from jax.experimental.pallas import tpu_sc as plsc

N = 10000
F = 128
H = 64
E = 320000
NC = 2
NS = 16
NP = 10240
RPT = NP // NS
EPT = E // (NC * NS)
K = 80
NCHUNK = EPT // K

_mesh = plsc.VectorSubcoreMesh(core_axis_name="c", subcore_axis_name="s")


@functools.partial(
    pl.kernel,
    mesh=_mesh,
    out_type=jax.ShapeDtypeStruct((NC * NP,), jnp.float32),
    compiler_params=pltpu.CompilerParams(use_tc_tiling_on_sc=False),
    scratch_types=[
        pltpu.VMEM((NCHUNK, K), jnp.int32),
        pltpu.VMEM((K,), jnp.float32),
        pltpu.VMEM((RPT,), jnp.float32),
        pltpu.VMEM_SHARED((NP,), jnp.float32),
    ],
)
def _deg_kernel(dst_hbm, zeros_hbm, ones_hbm, out_hbm, dst_v, ones_v, wb_v, acc_sh):
    cid = lax.axis_index("c")
    sid = lax.axis_index("s")
    pltpu.sync_copy(ones_hbm, ones_v)
    pltpu.sync_copy(zeros_hbm, wb_v)
    pltpu.sync_copy(wb_v, acc_sh.at[pl.ds(sid * RPT, RPT)])
    pltpu.sync_copy(dst_hbm.at[cid, sid], dst_v)
    plsc.subcore_barrier()

    def body(j, carry):
        pltpu.sync_copy(ones_v, acc_sh.at[dst_v.at[j]], add=True)
        return carry

    lax.fori_loop(0, NCHUNK, body, 0)
    plsc.subcore_barrier()
    pltpu.sync_copy(acc_sh.at[pl.ds(sid * RPT, RPT)], wb_v)
    pltpu.sync_copy(wb_v, out_hbm.at[pl.ds(cid * NP + sid * RPT, RPT)])


@functools.partial(
    pl.kernel,
    mesh=_mesh,
    out_type=jax.ShapeDtypeStruct((NC * NP, H), jnp.float32),
    compiler_params=pltpu.CompilerParams(use_tc_tiling_on_sc=False),
    scratch_types=[
        pltpu.VMEM((NCHUNK, K), jnp.int32),
        pltpu.VMEM((NCHUNK, K), jnp.int32),
        pltpu.VMEM((K, H), jnp.float32),
        pltpu.VMEM((RPT, H), jnp.float32),
        pltpu.VMEM_SHARED((NP, H), jnp.float32),
        pltpu.SemaphoreType.DMA,
    ],
)
def _agg_kernel(g_hbm, src_hbm, dst_hbm, zeros_hbm, out_hbm,
                src_v, dst_v, rows_v, wb_v, acc_sh, sem):
    cid = lax.axis_index("c")
    sid = lax.axis_index("s")
    pltpu.sync_copy(zeros_hbm, wb_v)
    pltpu.sync_copy(wb_v, acc_sh.at[pl.ds(sid * RPT, RPT)])
    pltpu.sync_copy(src_hbm.at[cid, sid], src_v)
    pltpu.sync_copy(dst_hbm.at[cid, sid], dst_v)
    plsc.subcore_barrier()

    def body(j, carry):
        pltpu.async_copy(g_hbm.at[src_v.at[j]], rows_v, sem).wait()
        pltpu.sync_copy(rows_v, acc_sh.at[dst_v.at[j]], add=True)
        return carry

    lax.fori_loop(0, NCHUNK, body, 0)
    plsc.subcore_barrier()
    pltpu.sync_copy(acc_sh.at[pl.ds(sid * RPT, RPT)], wb_v)
    pltpu.sync_copy(wb_v, out_hbm.at[pl.ds(cid * NP + sid * RPT, RPT)])


def _tc_first(d0_ref, d1_ref, x_ref, w1_ref, g_ref, dinv_ref):
    dinv = lax.rsqrt(d0_ref[...] + d1_ref[...] + 1.0)
    h = jnp.dot(x_ref[...], w1_ref[...], preferred_element_type=jnp.float32)
    dinv_ref[...] = dinv
    g_ref[...] = h * dinv


def _tc_mid(p0_ref, p1_ref, g_ref, dinv_ref, b_ref, w2_ref, g2_ref):
    dinv = dinv_ref[...]
    z = dinv * (p0_ref[...] + p1_ref[...] + g_ref[...]) + b_ref[...]
    z = jnp.maximum(z, 0.0)
    g2_ref[...] = jnp.dot(z, w2_ref[...], preferred_element_type=jnp.float32) * dinv


def _tc_last(p0_ref, p1_ref, g_ref, dinv_ref, b_ref, out_ref):
    z = dinv_ref[...] * (p0_ref[...] + p1_ref[...] + g_ref[...]) + b_ref[...]
    out_ref[...] = jnp.maximum(z, 0.0)


def kernel(x, edge_index, W1, b1, W2, b2):
    ei = edge_index.astype(jnp.int32)
    src = ei[0].reshape(NC, NS, NCHUNK, K)
    dst = ei[1].reshape(NC, NS, NCHUNK, K)

    zeros_row = jnp.zeros((RPT,), jnp.float32)
    ones_row = jnp.ones((K,), jnp.float32)
    zeros_blk = jnp.zeros((RPT, H), jnp.float32)

    deg = _deg_kernel(dst, zeros_row, ones_row)
    d0p = deg[:NP].reshape(NP, 1)
    d1p = deg[NP:].reshape(NP, 1)

    xp = jnp.concatenate([x, jnp.zeros((NP - N, F), x.dtype)], axis=0)
    b1r = b1.reshape(1, H)
    b2r = b2.reshape(1, H)

    g1, dinv = pl.pallas_call(
        _tc_first,
        out_shape=[
            jax.ShapeDtypeStruct((NP, H), jnp.float32),
            jax.ShapeDtypeStruct((NP, 1), jnp.float32),
        ],
    )(d0p, d1p, xp, W1)

    acc1 = _agg_kernel(g1, src, dst, zeros_blk)
    p10 = acc1[:NP]
    p11 = acc1[NP:]

    g2 = pl.pallas_call(
        _tc_mid,
        out_shape=jax.ShapeDtypeStruct((NP, H), jnp.float32),
    )(p10, p11, g1, dinv, b1r, W2)

    acc2 = _agg_kernel(g2, src, dst, zeros_blk)
    p20 = acc2[:NP]
    p21 = acc2[NP:]

    out = pl.pallas_call(
        _tc_last,
        out_shape=jax.ShapeDtypeStruct((NP, H), jnp.float32),
    )(p20, p21, g2, dinv, b2r)

    return out[:N]


# speedup vs baseline: 32.3294x; 1.4135x over previous
import functools

import jax
import jax.numpy as jnp
from jax import lax
from jax.experimental import pallas as pl
from jax.experimental.pallas import tpu as pltpu
from jax.experimental.pallas import tpu_sc as plsc

N = 10000
F = 128
H = 64
E = 320000
NC = 2
NS = 16
NP = 10240
RPT = NP // NS
EPT = E // (NC * NS)
K = 80
NCHUNK = EPT // K

_mesh = plsc.VectorSubcoreMesh(core_axis_name="c", subcore_axis_name="s")


@functools.partial(
    pl.kernel,
    mesh=_mesh,
    out_type=jax.ShapeDtypeStruct((NC * NP,), jnp.float32),
    compiler_params=pltpu.CompilerParams(use_tc_tiling_on_sc=False),
    scratch_types=[
        pltpu.VMEM((NCHUNK, K), jnp.int32),
        pltpu.VMEM((K,), jnp.float32),
        pltpu.VMEM((RPT,), jnp.float32),
        pltpu.VMEM_SHARED((NP,), jnp.float32),
    ],
)
def _deg_kernel(dst_hbm, zeros_hbm, ones_hbm, out_hbm, dst_v, ones_v, wb_v, acc_sh):
    cid = lax.axis_index("c")
    sid = lax.axis_index("s")
    pltpu.sync_copy(ones_hbm, ones_v)
    pltpu.sync_copy(zeros_hbm, wb_v)
    pltpu.sync_copy(wb_v, acc_sh.at[pl.ds(sid * RPT, RPT)])
    pltpu.sync_copy(dst_hbm.at[cid, sid], dst_v)
    plsc.subcore_barrier()

    def body(j, carry):
        pltpu.sync_copy(ones_v, acc_sh.at[dst_v.at[j]], add=True)
        return carry

    lax.fori_loop(0, NCHUNK, body, 0)
    plsc.subcore_barrier()
    pltpu.sync_copy(acc_sh.at[pl.ds(sid * RPT, RPT)], wb_v)
    pltpu.sync_copy(wb_v, out_hbm.at[pl.ds(cid * NP + sid * RPT, RPT)])


@functools.partial(
    pl.kernel,
    mesh=_mesh,
    out_type=jax.ShapeDtypeStruct((NC * NP, H), jnp.float32),
    compiler_params=pltpu.CompilerParams(use_tc_tiling_on_sc=False),
    scratch_types=[
        pltpu.VMEM((NCHUNK, K), jnp.int32),
        pltpu.VMEM((NCHUNK, K), jnp.int32),
        pltpu.VMEM((K, H), jnp.float32),
        pltpu.VMEM((K, H), jnp.float32),
        pltpu.VMEM((RPT, H), jnp.float32),
        pltpu.VMEM_SHARED((NP, H), jnp.float32),
        pltpu.SemaphoreType.DMA,
        pltpu.SemaphoreType.DMA,
    ],
)
def _agg_kernel(g_hbm, src_hbm, dst_hbm, zeros_hbm, out_hbm,
                src_v, dst_v, rows0_v, rows1_v, wb_v, acc_sh, sem0, sem1):
    cid = lax.axis_index("c")
    sid = lax.axis_index("s")
    pltpu.sync_copy(zeros_hbm, wb_v)
    pltpu.sync_copy(wb_v, acc_sh.at[pl.ds(sid * RPT, RPT)])
    pltpu.sync_copy(src_hbm.at[cid, sid], src_v)
    pltpu.sync_copy(dst_hbm.at[cid, sid], dst_v)
    plsc.subcore_barrier()

    bufs = ((rows0_v, sem0), (rows1_v, sem1))
    pltpu.async_copy(g_hbm.at[src_v.at[0]], rows0_v, sem0)
    pltpu.async_copy(g_hbm.at[src_v.at[1]], rows1_v, sem1)

    def body(i, carry):
        j2 = 2 * i
        for b in range(2):
            rows, sem = bufs[b]
            j = j2 + b
            pltpu.make_async_copy(g_hbm.at[src_v.at[j]], rows, sem).wait()
            pltpu.sync_copy(rows, acc_sh.at[dst_v.at[j]], add=True)

            @pl.when(j + 2 < NCHUNK)
            def _():
                pltpu.async_copy(g_hbm.at[src_v.at[j + 2]], rows, sem)

        return carry

    lax.fori_loop(0, (NCHUNK - 1) // 2, body, 0)
    pltpu.make_async_copy(
        g_hbm.at[src_v.at[NCHUNK - 1]], rows0_v, sem0).wait()
    pltpu.sync_copy(rows0_v, acc_sh.at[dst_v.at[NCHUNK - 1]], add=True)
    plsc.subcore_barrier()
    pltpu.sync_copy(acc_sh.at[pl.ds(sid * RPT, RPT)], wb_v)
    pltpu.sync_copy(wb_v, out_hbm.at[pl.ds(cid * NP + sid * RPT, RPT)])


def _tc_first(d0_ref, d1_ref, x_ref, w1_ref, g_ref, dinv_ref):
    dinv = lax.rsqrt(d0_ref[...] + d1_ref[...] + 1.0)
    h = jnp.dot(x_ref[...], w1_ref[...], preferred_element_type=jnp.float32)
    dinv_ref[...] = dinv
    g_ref[...] = h * dinv


def _tc_mid(p0_ref, p1_ref, g_ref, dinv_ref, b_ref, w2_ref, g2_ref):
    dinv = dinv_ref[...]
    z = dinv * (p0_ref[...] + p1_ref[...] + g_ref[...]) + b_ref[...]
    z = jnp.maximum(z, 0.0)
    g2_ref[...] = jnp.dot(z, w2_ref[...], preferred_element_type=jnp.float32) * dinv


def _tc_last(p0_ref, p1_ref, g_ref, dinv_ref, b_ref, out_ref):
    z = dinv_ref[...] * (p0_ref[...] + p1_ref[...] + g_ref[...]) + b_ref[...]
    out_ref[...] = jnp.maximum(z, 0.0)


def kernel(x, edge_index, W1, b1, W2, b2):
    ei = edge_index.astype(jnp.int32)
    src = ei[0].reshape(NC, NS, NCHUNK, K)
    dst = ei[1].reshape(NC, NS, NCHUNK, K)

    zeros_row = jnp.zeros((RPT,), jnp.float32)
    ones_row = jnp.ones((K,), jnp.float32)
    zeros_blk = jnp.zeros((RPT, H), jnp.float32)

    deg = _deg_kernel(dst, zeros_row, ones_row)
    d0p = deg[:NP].reshape(NP, 1)
    d1p = deg[NP:].reshape(NP, 1)

    xp = jnp.concatenate([x, jnp.zeros((NP - N, F), x.dtype)], axis=0)
    b1r = b1.reshape(1, H)
    b2r = b2.reshape(1, H)

    g1, dinv = pl.pallas_call(
        _tc_first,
        out_shape=[
            jax.ShapeDtypeStruct((NP, H), jnp.float32),
            jax.ShapeDtypeStruct((NP, 1), jnp.float32),
        ],
    )(d0p, d1p, xp, W1)

    acc1 = _agg_kernel(g1, src, dst, zeros_blk)
    p10 = acc1[:NP]
    p11 = acc1[NP:]

    g2 = pl.pallas_call(
        _tc_mid,
        out_shape=jax.ShapeDtypeStruct((NP, H), jnp.float32),
    )(p10, p11, g1, dinv, b1r, W2)

    acc2 = _agg_kernel(g2, src, dst, zeros_blk)
    p20 = acc2[:NP]
    p21 = acc2[NP:]

    out = pl.pallas_call(
        _tc_last,
        out_shape=jax.ShapeDtypeStruct((NP, H), jnp.float32),
    )(p20, p21, g2, dinv, b2r)

    return out[:N]
